# Initial kernel scaffold; baseline (speedup 1.0000x reference)
import jax
import jax.numpy as jnp
from jax.experimental import pallas as pl


def kernel(xyz, params):
    raise NotImplementedError("write your pallas kernel here")


# speedup vs baseline: 3.3594x; 3.3594x over previous
import functools

import jax
import jax.numpy as jnp
from jax.experimental import pallas as pl
from jax.experimental.pallas import tpu as pltpu


def _fps_body(xyz_ref, out_ref, *, npoint):
    x = xyz_ref[:, 0, :]
    y = xyz_ref[:, 1, :]
    z = xyz_ref[:, 2, :]
    b, n = x.shape
    iota = jax.lax.broadcasted_iota(jnp.int32, (b, n), 1)
    iota_np = jax.lax.broadcasted_iota(jnp.int32, (1, 1, npoint), 2)

    def body(i, carry):
        distance, farthest, acc = carry
        mask = (iota == farthest).astype(jnp.float32)
        cx = jnp.sum(x * mask, axis=1, keepdims=True)
        cy = jnp.sum(y * mask, axis=1, keepdims=True)
        cz = jnp.sum(z * mask, axis=1, keepdims=True)
        cen = jnp.concatenate([cx[:, None, :], cy[:, None, :], cz[:, None, :]],
                              axis=1)
        acc = jnp.where(iota_np == i, cen, acc)
        dx = x - cx
        dy = y - cy
        dz = z - cz
        d = dx * dx + dy * dy + dz * dz
        distance = jnp.minimum(distance, d)
        dmax = jnp.max(distance, axis=1, keepdims=True)
        farthest = jnp.min(jnp.where(distance == dmax, iota, n),
                           axis=1, keepdims=True)
        return distance, farthest, acc

    init = (jnp.full((b, n), 1e10, jnp.float32),
            jnp.zeros((b, 1), jnp.int32),
            jnp.zeros((b, 3, npoint), jnp.float32))
    _, _, acc = jax.lax.fori_loop(0, npoint, body, init)
    out_ref[...] = acc


def _fps(xyz_cn, npoint):
    b, _, n = xyz_cn.shape
    return pl.pallas_call(
        functools.partial(_fps_body, npoint=npoint),
        out_shape=jax.ShapeDtypeStruct((b, 3, npoint), jnp.float32),
    )(xyz_cn)


def _cumsum_lanes(m):
    r, n = m.shape
    lane = jax.lax.broadcasted_iota(jnp.int32, (r, n), 1)
    p = m
    sh = 1
    while sh < n:
        rolled = pltpu.roll(p, sh, 1)
        p = p + jnp.where(lane >= sh, rolled, 0.0)
        sh *= 2
    return p


def _group_body(new_ref, xyzcn_ref, feat_ref, out_ref, *, r2, k, c):
    nx = new_ref[0]
    xc = xyzcn_ref[0]
    feats = feat_ref[0]
    s_t = nx.shape[0]
    n = xc.shape[1]

    sq_n = xc[0:1] * xc[0:1] + xc[1:2] * xc[1:2] + xc[2:3] * xc[2:3]
    sq_s = jnp.sum(nx * nx, axis=1, keepdims=True)
    cross = jnp.dot(nx, xc, preferred_element_type=jnp.float32)
    sqd = (sq_s + sq_n) - 2.0 * cross

    m = (sqd <= r2).astype(jnp.float32)
    p = _cumsum_lanes(m)
    pmax = p[:, n - 1:n]

    jv = jax.lax.broadcasted_iota(jnp.int32, (1, k, 1), 1).astype(
        jnp.float32) + 1.0
    sel = jnp.where((p[:, None, :] == jv) & (m[:, None, :] > 0.0), 1.0, 0.0)
    g = jnp.dot(sel.reshape(s_t * k, n), feats,
                preferred_element_type=jnp.float32,
                precision=jax.lax.Precision.HIGHEST)
    g = g.reshape(s_t, k, c + 3)
    pm = pmax[:, :, None]
    fallback = jnp.where(pm > 0.0, g[:, 0:1, :], feats[n - 1:n, :][None])
    g = jnp.where(jv <= pm, g, fallback)
    rel = g[:, :, c:] - nx[:, None, :]
    out_ref[0] = jnp.concatenate([g[:, :, :c], rel], axis=2)


def _group(new_sn3, xyz_cn, feats_nc, radius, k, s_t):
    b, s, _ = new_sn3.shape
    n = xyz_cn.shape[2]
    cf = feats_nc.shape[2]
    c = cf - 3
    grid = (b, s // s_t)
    return pl.pallas_call(
        functools.partial(_group_body, r2=radius * radius, k=k, c=c),
        grid=grid,
        in_specs=[
            pl.BlockSpec((1, s_t, 3), lambda ib, js: (ib, js, 0)),
            pl.BlockSpec((1, 3, n), lambda ib, js: (ib, 0, 0)),
            pl.BlockSpec((1, n, cf), lambda ib, js: (ib, 0, 0)),
        ],
        out_specs=pl.BlockSpec((1, s_t, k, cf), lambda ib, js: (ib, js, 0, 0)),
        out_shape=jax.ShapeDtypeStruct((b, s, k, cf), jnp.float32),
    )(new_sn3, xyz_cn, feats_nc)


def _conv_body(x_ref, w_ref, bias_ref, a_ref, c_ref, y_ref, s1_ref, s2_ref,
               *, act):
    x = x_ref[0]
    s_t, k, ci = x.shape
    x2 = x.reshape(s_t * k, ci)
    if act:
        x2 = jnp.maximum(x2 * a_ref[...] + c_ref[...], 0.0)
    y = jnp.dot(x2, w_ref[...], preferred_element_type=jnp.float32)
    y = y + bias_ref[...]
    co = y.shape[1]
    y_ref[0] = y.reshape(s_t, k, co)
    ps1 = jnp.sum(y, axis=0, keepdims=True)
    ps2 = jnp.sum(y * y, axis=0, keepdims=True)

    first = jnp.logical_and(pl.program_id(0) == 0, pl.program_id(1) == 0)

    @pl.when(first)
    def _():
        s1_ref[...] = ps1
        s2_ref[...] = ps2

    @pl.when(jnp.logical_not(first))
    def _():
        s1_ref[...] += ps1
        s2_ref[...] += ps2


def _conv(x_bskc, wt, bias, a, c, act, s_t):
    b, s, k, ci = x_bskc.shape
    co = wt.shape[1]
    grid = (b, s // s_t)
    y, s1, s2 = pl.pallas_call(
        functools.partial(_conv_body, act=act),
        grid=grid,
        in_specs=[
            pl.BlockSpec((1, s_t, k, ci), lambda ib, js: (ib, js, 0, 0)),
            pl.BlockSpec((ci, co), lambda ib, js: (0, 0)),
            pl.BlockSpec((1, co), lambda ib, js: (0, 0)),
            pl.BlockSpec((1, ci), lambda ib, js: (0, 0)),
            pl.BlockSpec((1, ci), lambda ib, js: (0, 0)),
        ],
        out_specs=[
            pl.BlockSpec((1, s_t, k, co), lambda ib, js: (ib, js, 0, 0)),
            pl.BlockSpec((1, co), lambda ib, js: (0, 0)),
            pl.BlockSpec((1, co), lambda ib, js: (0, 0)),
        ],
        out_shape=[
            jax.ShapeDtypeStruct((b, s, k, co), jnp.float32),
            jax.ShapeDtypeStruct((1, co), jnp.float32),
            jax.ShapeDtypeStruct((1, co), jnp.float32),
        ],
        compiler_params=pltpu.CompilerParams(
            dimension_semantics=("arbitrary", "arbitrary")),
    )(x_bskc, wt, bias, a, c)
    return y, s1, s2


def _pool_body(y_ref, a_ref, c_ref, o_ref):
    y = y_ref[0]
    z = jnp.maximum(y * a_ref[...][None] + c_ref[...][None], 0.0)
    o_ref[0] = jnp.max(z, axis=1)


def _pool(y_bskc, a, c, s_t):
    b, s, k, co = y_bskc.shape
    grid = (b, s // s_t)
    return pl.pallas_call(
        _pool_body,
        grid=grid,
        in_specs=[
            pl.BlockSpec((1, s_t, k, co), lambda ib, js: (ib, js, 0, 0)),
            pl.BlockSpec((1, co), lambda ib, js: (0, 0)),
            pl.BlockSpec((1, co), lambda ib, js: (0, 0)),
        ],
        out_specs=pl.BlockSpec((1, s_t, co), lambda ib, js: (ib, js, 0)),
        out_shape=jax.ShapeDtypeStruct((b, s, co), jnp.float32),
    )(y_bskc, a, c)


_EPS = 1e-5


def _mlp_branch(grouped, layers, conv_s_t):
    b, s, k, _ = grouped.shape
    count = float(b * s * k)
    g = grouped
    a = jnp.ones((1, grouped.shape[3]), jnp.float32)
    c = jnp.zeros((1, grouped.shape[3]), jnp.float32)
    for li, (w, bias, gamma, beta) in enumerate(layers):
        y, s1, s2 = _conv(g, w.T, bias.reshape(1, -1), a, c,
                          act=(li > 0), s_t=conv_s_t)
        mean = s1 / count
        var = s2 / count - mean * mean
        inv = gamma.reshape(1, -1) * jax.lax.rsqrt(var + _EPS)
        a = inv
        c = beta.reshape(1, -1) - mean * inv
        g = y
    return _pool(g, a, c, conv_s_t)


def _sa_layer(xyz_cn, xyz_nc, points_nc, npoint, branches, group_s_t):
    new_c3 = _fps(xyz_cn, npoint)
    new_sn3 = jnp.transpose(new_c3, (0, 2, 1))
    feats = jnp.concatenate([points_nc, xyz_nc], axis=2)
    outs = []
    for radius, k, layers in branches:
        grouped = _group(new_sn3, xyz_cn, feats, radius, k, s_t=group_s_t)
        conv_s_t = max(1, 512 // k)
        if conv_s_t > npoint:
            conv_s_t = npoint
        outs.append(_mlp_branch(grouped, layers, conv_s_t))
    return new_c3, new_sn3, jnp.concatenate(outs, axis=2)


def kernel(xyz, params):
    xyz = xyz.astype(jnp.float32)
    xyz_nc = jnp.transpose(xyz, (0, 2, 1))

    sa1 = [(0.05, 16, params['sa1'][0]), (0.1, 32, params['sa1'][1])]
    l1_c3, l1_nc3, l1_points = _sa_layer(xyz, xyz_nc, xyz_nc, 512, sa1,
                                         group_s_t=8)

    sa2 = [(0.1, 16, params['sa2'][0]), (0.2, 32, params['sa2'][1])]
    l2_c3, _, l2_points = _sa_layer(l1_c3, l1_nc3, l1_points, 256, sa2,
                                    group_s_t=32)

    return l2_c3, jnp.transpose(l2_points, (0, 2, 1))


# speedup vs baseline: 3.5425x; 1.0545x over previous
import functools

import jax
import jax.numpy as jnp
from jax.experimental import pallas as pl
from jax.experimental.pallas import tpu as pltpu


def _fps_body(xyz_ref, out_ref, *, npoint):
    x = xyz_ref[:, 0, :]
    y = xyz_ref[:, 1, :]
    z = xyz_ref[:, 2, :]
    b, n = x.shape
    iota = jax.lax.broadcasted_iota(jnp.int32, (b, n), 1)
    iota_np = jax.lax.broadcasted_iota(jnp.int32, (1, 1, npoint), 2)

    def body(i, carry):
        distance, farthest, acc = carry
        mask = (iota == farthest).astype(jnp.float32)
        cx = jnp.sum(x * mask, axis=1, keepdims=True)
        cy = jnp.sum(y * mask, axis=1, keepdims=True)
        cz = jnp.sum(z * mask, axis=1, keepdims=True)
        cen = jnp.concatenate([cx[:, None, :], cy[:, None, :], cz[:, None, :]],
                              axis=1)
        acc = jnp.where(iota_np == i, cen, acc)
        dx = x - cx
        dy = y - cy
        dz = z - cz
        d = dx * dx + dy * dy + dz * dz
        distance = jnp.minimum(distance, d)
        dmax = jnp.max(distance, axis=1, keepdims=True)
        farthest = jnp.min(jnp.where(distance == dmax, iota, n),
                           axis=1, keepdims=True)
        return distance, farthest, acc

    init = (jnp.full((b, n), 1e10, jnp.float32),
            jnp.zeros((b, 1), jnp.int32),
            jnp.zeros((b, 3, npoint), jnp.float32))
    _, _, acc = jax.lax.fori_loop(0, npoint, body, init)
    out_ref[...] = acc


def _fps(xyz_cn, npoint):
    b, _, n = xyz_cn.shape
    return pl.pallas_call(
        functools.partial(_fps_body, npoint=npoint),
        out_shape=jax.ShapeDtypeStruct((b, 3, npoint), jnp.float32),
    )(xyz_cn)


def _cumsum_lanes(m):
    r, n = m.shape
    lane = jax.lax.broadcasted_iota(jnp.int32, (r, n), 1)
    p = m
    sh = 1
    while sh < n:
        rolled = pltpu.roll(p, sh, 1)
        p = p + jnp.where(lane >= sh, rolled, 0.0)
        sh *= 2
    return p


def _group_body(new_ref, xyzcn_ref, feat_ref, out_ref, *, r2, k, c):
    nx = new_ref[0]
    xc = xyzcn_ref[0]
    feats = feat_ref[0]
    s_t = nx.shape[0]
    n = xc.shape[1]

    sq_n = xc[0:1] * xc[0:1] + xc[1:2] * xc[1:2] + xc[2:3] * xc[2:3]
    sq_s = jnp.sum(nx * nx, axis=1, keepdims=True)
    cross = jnp.dot(nx, xc, preferred_element_type=jnp.float32)
    sqd = (sq_s + sq_n) - 2.0 * cross

    m = (sqd <= r2).astype(jnp.float32)
    p = _cumsum_lanes(m)
    pmax = p[:, n - 1:n]

    jv = jax.lax.broadcasted_iota(jnp.int32, (1, k, 1), 1).astype(
        jnp.float32) + 1.0
    q = p * m
    sel = (q[:, None, :] == jv).astype(jnp.float32)
    g = jnp.dot(sel.reshape(s_t * k, n), feats,
                preferred_element_type=jnp.float32,
                precision=jax.lax.Precision.HIGHEST)
    g = g.reshape(s_t, k, c + 3)
    pm = pmax[:, :, None]
    fallback = jnp.where(pm > 0.0, g[:, 0:1, :], feats[n - 1:n, :][None])
    g = jnp.where(jv <= pm, g, fallback)
    rel = g[:, :, c:] - nx[:, None, :]
    out_ref[0] = jnp.concatenate([g[:, :, :c], rel], axis=2)


def _group(new_sn3, xyz_cn, feats_nc, radius, k, s_t):
    b, s, _ = new_sn3.shape
    n = xyz_cn.shape[2]
    cf = feats_nc.shape[2]
    c = cf - 3
    grid = (b, s // s_t)
    return pl.pallas_call(
        functools.partial(_group_body, r2=radius * radius, k=k, c=c),
        grid=grid,
        in_specs=[
            pl.BlockSpec((1, s_t, 3), lambda ib, js: (ib, js, 0)),
            pl.BlockSpec((1, 3, n), lambda ib, js: (ib, 0, 0)),
            pl.BlockSpec((1, n, cf), lambda ib, js: (ib, 0, 0)),
        ],
        out_specs=pl.BlockSpec((1, s_t, k, cf), lambda ib, js: (ib, js, 0, 0)),
        out_shape=jax.ShapeDtypeStruct((b, s, k, cf), jnp.float32),
    )(new_sn3, xyz_cn, feats_nc)


def _conv_body(x_ref, w_ref, bias_ref, a_ref, c_ref, y_ref, s1_ref, s2_ref,
               *, act):
    x = x_ref[0]
    s_t, k, ci = x.shape
    x2 = x.reshape(s_t * k, ci)
    if act:
        x2 = jnp.maximum(x2 * a_ref[...] + c_ref[...], 0.0)
    y = jnp.dot(x2, w_ref[...], preferred_element_type=jnp.float32)
    y = y + bias_ref[...]
    co = y.shape[1]
    y_ref[0] = y.reshape(s_t, k, co)
    ps1 = jnp.sum(y, axis=0, keepdims=True)
    ps2 = jnp.sum(y * y, axis=0, keepdims=True)

    first = jnp.logical_and(pl.program_id(0) == 0, pl.program_id(1) == 0)

    @pl.when(first)
    def _():
        s1_ref[...] = ps1
        s2_ref[...] = ps2

    @pl.when(jnp.logical_not(first))
    def _():
        s1_ref[...] += ps1
        s2_ref[...] += ps2


def _conv(x_bskc, wt, bias, a, c, act, s_t):
    b, s, k, ci = x_bskc.shape
    co = wt.shape[1]
    grid = (b, s // s_t)
    y, s1, s2 = pl.pallas_call(
        functools.partial(_conv_body, act=act),
        grid=grid,
        in_specs=[
            pl.BlockSpec((1, s_t, k, ci), lambda ib, js: (ib, js, 0, 0)),
            pl.BlockSpec((ci, co), lambda ib, js: (0, 0)),
            pl.BlockSpec((1, co), lambda ib, js: (0, 0)),
            pl.BlockSpec((1, ci), lambda ib, js: (0, 0)),
            pl.BlockSpec((1, ci), lambda ib, js: (0, 0)),
        ],
        out_specs=[
            pl.BlockSpec((1, s_t, k, co), lambda ib, js: (ib, js, 0, 0)),
            pl.BlockSpec((1, co), lambda ib, js: (0, 0)),
            pl.BlockSpec((1, co), lambda ib, js: (0, 0)),
        ],
        out_shape=[
            jax.ShapeDtypeStruct((b, s, k, co), jnp.float32),
            jax.ShapeDtypeStruct((1, co), jnp.float32),
            jax.ShapeDtypeStruct((1, co), jnp.float32),
        ],
        compiler_params=pltpu.CompilerParams(
            dimension_semantics=("arbitrary", "arbitrary")),
    )(x_bskc, wt, bias, a, c)
    return y, s1, s2


def _pool_body(y_ref, a_ref, c_ref, o_ref):
    y = y_ref[0]
    z = jnp.maximum(y * a_ref[...][None] + c_ref[...][None], 0.0)
    o_ref[0] = jnp.max(z, axis=1)


def _pool(y_bskc, a, c, s_t):
    b, s, k, co = y_bskc.shape
    grid = (b, s // s_t)
    return pl.pallas_call(
        _pool_body,
        grid=grid,
        in_specs=[
            pl.BlockSpec((1, s_t, k, co), lambda ib, js: (ib, js, 0, 0)),
            pl.BlockSpec((1, co), lambda ib, js: (0, 0)),
            pl.BlockSpec((1, co), lambda ib, js: (0, 0)),
        ],
        out_specs=pl.BlockSpec((1, s_t, co), lambda ib, js: (ib, js, 0)),
        out_shape=jax.ShapeDtypeStruct((b, s, co), jnp.float32),
    )(y_bskc, a, c)


_EPS = 1e-5


def _mlp_branch(grouped, layers, conv_s_t):
    b, s, k, _ = grouped.shape
    count = float(b * s * k)
    g = grouped
    a = jnp.ones((1, grouped.shape[3]), jnp.float32)
    c = jnp.zeros((1, grouped.shape[3]), jnp.float32)
    for li, (w, bias, gamma, beta) in enumerate(layers):
        y, s1, s2 = _conv(g, w.T, bias.reshape(1, -1), a, c,
                          act=(li > 0), s_t=conv_s_t)
        mean = s1 / count
        var = s2 / count - mean * mean
        inv = gamma.reshape(1, -1) * jax.lax.rsqrt(var + _EPS)
        a = inv
        c = beta.reshape(1, -1) - mean * inv
        g = y
    return _pool(g, a, c, conv_s_t)


def _sa_layer(xyz_cn, xyz_nc, points_nc, npoint, branches, group_s_t):
    new_c3 = _fps(xyz_cn, npoint)
    new_sn3 = jnp.transpose(new_c3, (0, 2, 1))
    feats = jnp.concatenate([points_nc, xyz_nc], axis=2)
    outs = []
    for radius, k, layers in branches:
        grouped = _group(new_sn3, xyz_cn, feats, radius, k, s_t=group_s_t)
        conv_s_t = max(1, 512 // k)
        if conv_s_t > npoint:
            conv_s_t = npoint
        outs.append(_mlp_branch(grouped, layers, conv_s_t))
    return new_c3, new_sn3, jnp.concatenate(outs, axis=2)


def kernel(xyz, params):
    xyz = xyz.astype(jnp.float32)
    xyz_nc = jnp.transpose(xyz, (0, 2, 1))

    sa1 = [(0.05, 16, params['sa1'][0]), (0.1, 32, params['sa1'][1])]
    l1_c3, l1_nc3, l1_points = _sa_layer(xyz, xyz_nc, xyz_nc, 512, sa1,
                                         group_s_t=8)

    sa2 = [(0.1, 16, params['sa2'][0]), (0.2, 32, params['sa2'][1])]
    l2_c3, _, l2_points = _sa_layer(l1_c3, l1_nc3, l1_points, 256, sa2,
                                    group_s_t=32)

    return l2_c3, jnp.transpose(l2_points, (0, 2, 1))


# speedup vs baseline: 4.7975x; 1.3543x over previous
import functools

import jax
import jax.numpy as jnp
from jax.experimental import pallas as pl
from jax.experimental.pallas import tpu as pltpu


def _fps_body(xyz_ref, out_ref, *, npoint):
    x = xyz_ref[:, 0, :]
    y = xyz_ref[:, 1, :]
    z = xyz_ref[:, 2, :]
    b, n = x.shape
    iota = jax.lax.broadcasted_iota(jnp.int32, (b, n), 1)
    iota_np = jax.lax.broadcasted_iota(jnp.int32, (1, 1, npoint), 2)

    def body(i, carry):
        distance, farthest, acc = carry
        mask = (iota == farthest).astype(jnp.float32)
        cx = jnp.sum(x * mask, axis=1, keepdims=True)
        cy = jnp.sum(y * mask, axis=1, keepdims=True)
        cz = jnp.sum(z * mask, axis=1, keepdims=True)
        cen = jnp.concatenate([cx[:, None, :], cy[:, None, :], cz[:, None, :]],
                              axis=1)
        acc = jnp.where(iota_np == i, cen, acc)
        dx = x - cx
        dy = y - cy
        dz = z - cz
        d = dx * dx + dy * dy + dz * dz
        distance = jnp.minimum(distance, d)
        dmax = jnp.max(distance, axis=1, keepdims=True)
        farthest = jnp.min(jnp.where(distance == dmax, iota, n),
                           axis=1, keepdims=True)
        return distance, farthest, acc

    init = (jnp.full((b, n), 1e10, jnp.float32),
            jnp.zeros((b, 1), jnp.int32),
            jnp.zeros((b, 3, npoint), jnp.float32))
    _, _, acc = jax.lax.fori_loop(0, npoint, body, init)
    out_ref[...] = acc


def _fps(xyz_cn, npoint):
    b, _, n = xyz_cn.shape
    return pl.pallas_call(
        functools.partial(_fps_body, npoint=npoint),
        out_shape=jax.ShapeDtypeStruct((b, 3, npoint), jnp.float32),
    )(xyz_cn)


def _cumsum_lanes(m):
    r, n = m.shape
    lane = jax.lax.broadcasted_iota(jnp.int32, (r, n), 1)
    p = m
    sh = 1
    while sh < n:
        rolled = pltpu.roll(p, sh, 1)
        p = p + jnp.where(lane >= sh, rolled, 0.0)
        sh *= 2
    return p


def _gc_body(new_ref, xyzcn_ref, feat_ref, w1_ref, b1_ref, w2_ref, b2_ref,
             y1_ref, y2_ref, p11_ref, p12_ref, p21_ref, p22_ref,
             *, r2s, ks, c):
    nx = new_ref[0]
    xc = xyzcn_ref[0]
    feats = feat_ref[0]
    s_t = nx.shape[0]
    n = xc.shape[1]

    sq_n = xc[0:1] * xc[0:1] + xc[1:2] * xc[1:2] + xc[2:3] * xc[2:3]
    sq_s = jnp.sum(nx * nx, axis=1, keepdims=True)
    cross = jnp.dot(nx, xc, preferred_element_type=jnp.float32)
    sqd = (sq_s + sq_n) - 2.0 * cross

    fh = feats.astype(jnp.bfloat16)
    rem = feats - fh.astype(jnp.float32)
    fm = rem.astype(jnp.bfloat16)
    fl = (rem - fm.astype(jnp.float32)).astype(jnp.bfloat16)

    first = jnp.logical_and(pl.program_id(0) == 0, pl.program_id(1) == 0)

    for r2, k, w_ref, bias_ref, y_ref, s1_ref, s2_ref in (
            (r2s[0], ks[0], w1_ref, b1_ref, y1_ref, p11_ref, p12_ref),
            (r2s[1], ks[1], w2_ref, b2_ref, y2_ref, p21_ref, p22_ref)):
        m = (sqd <= r2).astype(jnp.float32)
        p = _cumsum_lanes(m)
        pmax = p[:, n - 1:n]
        jv = jax.lax.broadcasted_iota(jnp.int32, (1, k, 1), 1).astype(
            jnp.float32) + 1.0
        q = p * m
        sel = (q[:, None, :] == jv).astype(jnp.bfloat16).reshape(s_t * k, n)
        g = (jnp.dot(sel, fh, preferred_element_type=jnp.float32)
             + jnp.dot(sel, fm, preferred_element_type=jnp.float32)
             + jnp.dot(sel, fl, preferred_element_type=jnp.float32))
        g = g.reshape(s_t, k, c + 3)
        pm = pmax[:, :, None]
        fallback = jnp.where(pm > 0.0, g[:, 0:1, :], feats[n - 1:n, :][None])
        g = jnp.where(jv <= pm, g, fallback)
        rel = g[:, :, c:] - nx[:, None, :]
        grouped = jnp.concatenate([g[:, :, :c], rel], axis=2)

        x2 = grouped.reshape(s_t * k, c + 3)
        y = jnp.dot(x2, w_ref[...], preferred_element_type=jnp.float32)
        y = y + bias_ref[...]
        co = y.shape[1]
        y_ref[0] = y.reshape(s_t, k, co)
        ps1 = jnp.sum(y, axis=0, keepdims=True)
        ps2 = jnp.sum(y * y, axis=0, keepdims=True)

        @pl.when(first)
        def _():
            s1_ref[...] = ps1
            s2_ref[...] = ps2

        @pl.when(jnp.logical_not(first))
        def _():
            s1_ref[...] += ps1
            s2_ref[...] += ps2


def _group_conv1(new_sn3, xyz_cn, feats_nc, branches, s_t):
    b, s, _ = new_sn3.shape
    n = xyz_cn.shape[2]
    cf = feats_nc.shape[2]
    c = cf - 3
    (r1, k1, l1), (r2b, k2, l2) = branches
    w1t, b1 = l1[0][0].T, l1[0][1].reshape(1, -1)
    w2t, b2 = l2[0][0].T, l2[0][1].reshape(1, -1)
    co1, co2 = w1t.shape[1], w2t.shape[1]
    grid = (b, s // s_t)
    const = lambda ib, js: (0, 0)
    return pl.pallas_call(
        functools.partial(_gc_body, r2s=(r1 * r1, r2b * r2b), ks=(k1, k2),
                          c=c),
        grid=grid,
        in_specs=[
            pl.BlockSpec((1, s_t, 3), lambda ib, js: (ib, js, 0)),
            pl.BlockSpec((1, 3, n), lambda ib, js: (ib, 0, 0)),
            pl.BlockSpec((1, n, cf), lambda ib, js: (ib, 0, 0)),
            pl.BlockSpec((cf, co1), const),
            pl.BlockSpec((1, co1), const),
            pl.BlockSpec((cf, co2), const),
            pl.BlockSpec((1, co2), const),
        ],
        out_specs=[
            pl.BlockSpec((1, s_t, k1, co1), lambda ib, js: (ib, js, 0, 0)),
            pl.BlockSpec((1, s_t, k2, co2), lambda ib, js: (ib, js, 0, 0)),
            pl.BlockSpec((1, co1), const),
            pl.BlockSpec((1, co1), const),
            pl.BlockSpec((1, co2), const),
            pl.BlockSpec((1, co2), const),
        ],
        out_shape=[
            jax.ShapeDtypeStruct((b, s, k1, co1), jnp.float32),
            jax.ShapeDtypeStruct((b, s, k2, co2), jnp.float32),
            jax.ShapeDtypeStruct((1, co1), jnp.float32),
            jax.ShapeDtypeStruct((1, co1), jnp.float32),
            jax.ShapeDtypeStruct((1, co2), jnp.float32),
            jax.ShapeDtypeStruct((1, co2), jnp.float32),
        ],
        compiler_params=pltpu.CompilerParams(
            dimension_semantics=("arbitrary", "arbitrary")),
    )(new_sn3, xyz_cn, feats_nc, w1t, b1, w2t, b2)


def _conv_body(x_ref, w_ref, bias_ref, a_ref, c_ref, y_ref, s1_ref, s2_ref,
               *, act):
    x = x_ref[0]
    s_t, k, ci = x.shape
    x2 = x.reshape(s_t * k, ci)
    if act:
        x2 = jnp.maximum(x2 * a_ref[...] + c_ref[...], 0.0)
    y = jnp.dot(x2, w_ref[...], preferred_element_type=jnp.float32)
    y = y + bias_ref[...]
    co = y.shape[1]
    y_ref[0] = y.reshape(s_t, k, co)
    ps1 = jnp.sum(y, axis=0, keepdims=True)
    ps2 = jnp.sum(y * y, axis=0, keepdims=True)

    first = jnp.logical_and(pl.program_id(0) == 0, pl.program_id(1) == 0)

    @pl.when(first)
    def _():
        s1_ref[...] = ps1
        s2_ref[...] = ps2

    @pl.when(jnp.logical_not(first))
    def _():
        s1_ref[...] += ps1
        s2_ref[...] += ps2


def _conv(x_bskc, wt, bias, a, c, act, s_t):
    b, s, k, ci = x_bskc.shape
    co = wt.shape[1]
    grid = (b, s // s_t)
    y, s1, s2 = pl.pallas_call(
        functools.partial(_conv_body, act=act),
        grid=grid,
        in_specs=[
            pl.BlockSpec((1, s_t, k, ci), lambda ib, js: (ib, js, 0, 0)),
            pl.BlockSpec((ci, co), lambda ib, js: (0, 0)),
            pl.BlockSpec((1, co), lambda ib, js: (0, 0)),
            pl.BlockSpec((1, ci), lambda ib, js: (0, 0)),
            pl.BlockSpec((1, ci), lambda ib, js: (0, 0)),
        ],
        out_specs=[
            pl.BlockSpec((1, s_t, k, co), lambda ib, js: (ib, js, 0, 0)),
            pl.BlockSpec((1, co), lambda ib, js: (0, 0)),
            pl.BlockSpec((1, co), lambda ib, js: (0, 0)),
        ],
        out_shape=[
            jax.ShapeDtypeStruct((b, s, k, co), jnp.float32),
            jax.ShapeDtypeStruct((1, co), jnp.float32),
            jax.ShapeDtypeStruct((1, co), jnp.float32),
        ],
        compiler_params=pltpu.CompilerParams(
            dimension_semantics=("arbitrary", "arbitrary")),
    )(x_bskc, wt, bias, a, c)
    return y, s1, s2


def _pool_body(y_ref, a_ref, c_ref, o_ref):
    y = y_ref[0]
    z = jnp.maximum(y * a_ref[...][None] + c_ref[...][None], 0.0)
    o_ref[0] = jnp.max(z, axis=1)


def _pool(y_bskc, a, c, s_t):
    b, s, k, co = y_bskc.shape
    grid = (b, s // s_t)
    return pl.pallas_call(
        _pool_body,
        grid=grid,
        in_specs=[
            pl.BlockSpec((1, s_t, k, co), lambda ib, js: (ib, js, 0, 0)),
            pl.BlockSpec((1, co), lambda ib, js: (0, 0)),
            pl.BlockSpec((1, co), lambda ib, js: (0, 0)),
        ],
        out_specs=pl.BlockSpec((1, s_t, co), lambda ib, js: (ib, js, 0)),
        out_shape=jax.ShapeDtypeStruct((b, s, co), jnp.float32),
    )(y_bskc, a, c)


_EPS = 1e-5


def _bn_affine(s1, s2, gamma, beta, count):
    mean = s1 / count
    var = s2 / count - mean * mean
    inv = gamma.reshape(1, -1) * jax.lax.rsqrt(var + _EPS)
    return inv, beta.reshape(1, -1) - mean * inv


def _mlp_branch(y1, s1, s2, layers, conv_s_t):
    b, s, k, _ = y1.shape
    count = float(b * s * k)
    a, c = _bn_affine(s1, s2, layers[0][2], layers[0][3], count)
    g = y1
    for (w, bias, gamma, beta) in layers[1:]:
        y, t1, t2 = _conv(g, w.T, bias.reshape(1, -1), a, c,
                          act=True, s_t=conv_s_t)
        a, c = _bn_affine(t1, t2, gamma, beta, count)
        g = y
    return _pool(g, a, c, conv_s_t)


def _sa_layer(xyz_cn, xyz_nc, points_nc, npoint, branches, group_s_t):
    new_c3 = _fps(xyz_cn, npoint)
    new_sn3 = jnp.transpose(new_c3, (0, 2, 1))
    feats = jnp.concatenate([points_nc, xyz_nc], axis=2)
    y1a, y1b, s1a, s2a, s1b, s2b = _group_conv1(new_sn3, xyz_cn, feats,
                                                branches, s_t=group_s_t)
    outs = []
    for (radius, k, layers), y1, s1, s2 in (
            (branches[0], y1a, s1a, s2a), (branches[1], y1b, s1b, s2b)):
        conv_s_t = max(1, 512 // k)
        if conv_s_t > npoint:
            conv_s_t = npoint
        outs.append(_mlp_branch(y1, s1, s2, layers, conv_s_t))
    return new_c3, new_sn3, jnp.concatenate(outs, axis=2)


def kernel(xyz, params):
    xyz = xyz.astype(jnp.float32)
    xyz_nc = jnp.transpose(xyz, (0, 2, 1))

    sa1 = [(0.05, 16, params['sa1'][0]), (0.1, 32, params['sa1'][1])]
    l1_c3, l1_nc3, l1_points = _sa_layer(xyz, xyz_nc, xyz_nc, 512, sa1,
                                         group_s_t=8)

    sa2 = [(0.1, 16, params['sa2'][0]), (0.2, 32, params['sa2'][1])]
    l2_c3, _, l2_points = _sa_layer(l1_c3, l1_nc3, l1_points, 256, sa2,
                                    group_s_t=32)

    return l2_c3, jnp.transpose(l2_points, (0, 2, 1))


# speedup vs baseline: 6.3728x; 1.3284x over previous
import functools

import jax
import jax.numpy as jnp
from jax.experimental import pallas as pl
from jax.experimental.pallas import tpu as pltpu


def _fps_body(xyz_ref, out_ref, *, npoint):
    x = xyz_ref[:, 0, :]
    y = xyz_ref[:, 1, :]
    z = xyz_ref[:, 2, :]
    b, n = x.shape
    iota = jax.lax.broadcasted_iota(jnp.int32, (b, n), 1)
    iota_np = jax.lax.broadcasted_iota(jnp.int32, (1, 1, npoint), 2)

    def body(i, carry):
        distance, farthest, acc = carry
        mask = (iota == farthest).astype(jnp.float32)
        cx = jnp.sum(x * mask, axis=1, keepdims=True)
        cy = jnp.sum(y * mask, axis=1, keepdims=True)
        cz = jnp.sum(z * mask, axis=1, keepdims=True)
        cen = jnp.concatenate([cx[:, None, :], cy[:, None, :], cz[:, None, :]],
                              axis=1)
        acc = jnp.where(iota_np == i, cen, acc)
        dx = x - cx
        dy = y - cy
        dz = z - cz
        d = dx * dx + dy * dy + dz * dz
        distance = jnp.minimum(distance, d)
        dmax = jnp.max(distance, axis=1, keepdims=True)
        farthest = jnp.min(jnp.where(distance == dmax, iota, n),
                           axis=1, keepdims=True)
        return distance, farthest, acc

    init = (jnp.full((b, n), 1e10, jnp.float32),
            jnp.zeros((b, 1), jnp.int32),
            jnp.zeros((b, 3, npoint), jnp.float32))
    _, _, acc = jax.lax.fori_loop(0, npoint, body, init)
    out_ref[...] = acc


def _fps(xyz_cn, npoint):
    b, _, n = xyz_cn.shape
    return pl.pallas_call(
        functools.partial(_fps_body, npoint=npoint),
        out_shape=jax.ShapeDtypeStruct((b, 3, npoint), jnp.float32),
    )(xyz_cn)


def _cumsum_lanes(m):
    r, n = m.shape
    lane = jax.lax.broadcasted_iota(jnp.int32, (r, n), 1)
    p = m
    sh = 1
    while sh < n:
        rolled = pltpu.roll(p, sh, 1)
        p = p + jnp.where(lane >= sh, rolled, 0.0)
        sh *= 2
    return p


def _gc_body(new_ref, xyzcn_ref, feat_ref, w1_ref, b1_ref, w2_ref, b2_ref,
             y1_ref, y2_ref, p11_ref, p12_ref, p21_ref, p22_ref,
             *, r2s, ks, c):
    nx = new_ref[0]
    xc = xyzcn_ref[0]
    feats = feat_ref[0]
    s_t = nx.shape[0]
    n = xc.shape[1]

    sq_n = xc[0:1] * xc[0:1] + xc[1:2] * xc[1:2] + xc[2:3] * xc[2:3]
    sq_s = jnp.sum(nx * nx, axis=1, keepdims=True)
    cross = jnp.dot(nx, xc, preferred_element_type=jnp.float32)
    sqd = (sq_s + sq_n) - 2.0 * cross

    fh = feats.astype(jnp.bfloat16)
    rem = feats - fh.astype(jnp.float32)
    fm = rem.astype(jnp.bfloat16)
    fl = (rem - fm.astype(jnp.float32)).astype(jnp.bfloat16)

    first = jnp.logical_and(pl.program_id(0) == 0, pl.program_id(1) == 0)

    for r2, k, w_ref, bias_ref, y_ref, s1_ref, s2_ref in (
            (r2s[0], ks[0], w1_ref, b1_ref, y1_ref, p11_ref, p12_ref),
            (r2s[1], ks[1], w2_ref, b2_ref, y2_ref, p21_ref, p22_ref)):
        m = (sqd <= r2).astype(jnp.float32)
        p = _cumsum_lanes(m)
        pmax = p[:, n - 1:n]
        jv = jax.lax.broadcasted_iota(jnp.int32, (1, k, 1), 1).astype(
            jnp.float32) + 1.0
        q = jnp.minimum(p * m, float(k + 1)).astype(jnp.bfloat16)
        jvb = jv.astype(jnp.bfloat16)
        one = jnp.ones((), jnp.bfloat16)
        zero = jnp.zeros((), jnp.bfloat16)
        sel = jnp.where(q[:, None, :] == jvb, one, zero).reshape(s_t * k, n)
        g = (jnp.dot(sel, fh, preferred_element_type=jnp.float32)
             + jnp.dot(sel, fm, preferred_element_type=jnp.float32)
             + jnp.dot(sel, fl, preferred_element_type=jnp.float32))
        g = g.reshape(s_t, k, c + 3)
        pm = pmax[:, :, None]
        fallback = jnp.where(pm > 0.0, g[:, 0:1, :], feats[n - 1:n, :][None])
        g = jnp.where(jv <= pm, g, fallback)
        rel = g[:, :, c:] - nx[:, None, :]
        grouped = jnp.concatenate([g[:, :, :c], rel], axis=2)

        x2 = grouped.reshape(s_t * k, c + 3)
        y = jnp.dot(x2, w_ref[...], preferred_element_type=jnp.float32)
        y = y + bias_ref[...]
        co = y.shape[1]
        y_ref[0] = y.reshape(s_t, k, co)
        ps1 = jnp.sum(y, axis=0, keepdims=True)
        ps2 = jnp.sum(y * y, axis=0, keepdims=True)

        @pl.when(first)
        def _():
            s1_ref[...] = ps1
            s2_ref[...] = ps2

        @pl.when(jnp.logical_not(first))
        def _():
            s1_ref[...] += ps1
            s2_ref[...] += ps2


def _group_conv1(new_sn3, xyz_cn, feats_nc, branches, s_t):
    b, s, _ = new_sn3.shape
    n = xyz_cn.shape[2]
    cf = feats_nc.shape[2]
    c = cf - 3
    (r1, k1, l1), (r2b, k2, l2) = branches
    w1t, b1 = l1[0][0].T, l1[0][1].reshape(1, -1)
    w2t, b2 = l2[0][0].T, l2[0][1].reshape(1, -1)
    co1, co2 = w1t.shape[1], w2t.shape[1]
    grid = (b, s // s_t)
    const = lambda ib, js: (0, 0)
    return pl.pallas_call(
        functools.partial(_gc_body, r2s=(r1 * r1, r2b * r2b), ks=(k1, k2),
                          c=c),
        grid=grid,
        in_specs=[
            pl.BlockSpec((1, s_t, 3), lambda ib, js: (ib, js, 0)),
            pl.BlockSpec((1, 3, n), lambda ib, js: (ib, 0, 0)),
            pl.BlockSpec((1, n, cf), lambda ib, js: (ib, 0, 0)),
            pl.BlockSpec((cf, co1), const),
            pl.BlockSpec((1, co1), const),
            pl.BlockSpec((cf, co2), const),
            pl.BlockSpec((1, co2), const),
        ],
        out_specs=[
            pl.BlockSpec((1, s_t, k1, co1), lambda ib, js: (ib, js, 0, 0)),
            pl.BlockSpec((1, s_t, k2, co2), lambda ib, js: (ib, js, 0, 0)),
            pl.BlockSpec((1, co1), const),
            pl.BlockSpec((1, co1), const),
            pl.BlockSpec((1, co2), const),
            pl.BlockSpec((1, co2), const),
        ],
        out_shape=[
            jax.ShapeDtypeStruct((b, s, k1, co1), jnp.float32),
            jax.ShapeDtypeStruct((b, s, k2, co2), jnp.float32),
            jax.ShapeDtypeStruct((1, co1), jnp.float32),
            jax.ShapeDtypeStruct((1, co1), jnp.float32),
            jax.ShapeDtypeStruct((1, co2), jnp.float32),
            jax.ShapeDtypeStruct((1, co2), jnp.float32),
        ],
        compiler_params=pltpu.CompilerParams(
            dimension_semantics=("arbitrary", "arbitrary")),
    )(new_sn3, xyz_cn, feats_nc, w1t, b1, w2t, b2)


def _conv_body(x_ref, w_ref, bias_ref, a_ref, c_ref, y_ref, s1_ref, s2_ref,
               *, act):
    x = x_ref[0]
    s_t, k, ci = x.shape
    x2 = x.reshape(s_t * k, ci)
    if act:
        x2 = jnp.maximum(x2 * a_ref[...] + c_ref[...], 0.0)
    y = jnp.dot(x2, w_ref[...], preferred_element_type=jnp.float32)
    y = y + bias_ref[...]
    co = y.shape[1]
    y_ref[0] = y.reshape(s_t, k, co)
    ps1 = jnp.sum(y, axis=0, keepdims=True)
    ps2 = jnp.sum(y * y, axis=0, keepdims=True)

    first = jnp.logical_and(pl.program_id(0) == 0, pl.program_id(1) == 0)

    @pl.when(first)
    def _():
        s1_ref[...] = ps1
        s2_ref[...] = ps2

    @pl.when(jnp.logical_not(first))
    def _():
        s1_ref[...] += ps1
        s2_ref[...] += ps2


def _conv(x_bskc, wt, bias, a, c, act, s_t):
    b, s, k, ci = x_bskc.shape
    co = wt.shape[1]
    grid = (b, s // s_t)
    y, s1, s2 = pl.pallas_call(
        functools.partial(_conv_body, act=act),
        grid=grid,
        in_specs=[
            pl.BlockSpec((1, s_t, k, ci), lambda ib, js: (ib, js, 0, 0)),
            pl.BlockSpec((ci, co), lambda ib, js: (0, 0)),
            pl.BlockSpec((1, co), lambda ib, js: (0, 0)),
            pl.BlockSpec((1, ci), lambda ib, js: (0, 0)),
            pl.BlockSpec((1, ci), lambda ib, js: (0, 0)),
        ],
        out_specs=[
            pl.BlockSpec((1, s_t, k, co), lambda ib, js: (ib, js, 0, 0)),
            pl.BlockSpec((1, co), lambda ib, js: (0, 0)),
            pl.BlockSpec((1, co), lambda ib, js: (0, 0)),
        ],
        out_shape=[
            jax.ShapeDtypeStruct((b, s, k, co), jnp.float32),
            jax.ShapeDtypeStruct((1, co), jnp.float32),
            jax.ShapeDtypeStruct((1, co), jnp.float32),
        ],
        compiler_params=pltpu.CompilerParams(
            dimension_semantics=("arbitrary", "arbitrary")),
    )(x_bskc, wt, bias, a, c)
    return y, s1, s2


def _pool_body(y_ref, a_ref, c_ref, o_ref):
    y = y_ref[0]
    z = jnp.maximum(y * a_ref[...][None] + c_ref[...][None], 0.0)
    o_ref[0] = jnp.max(z, axis=1)


def _pool(y_bskc, a, c, s_t):
    b, s, k, co = y_bskc.shape
    grid = (b, s // s_t)
    return pl.pallas_call(
        _pool_body,
        grid=grid,
        in_specs=[
            pl.BlockSpec((1, s_t, k, co), lambda ib, js: (ib, js, 0, 0)),
            pl.BlockSpec((1, co), lambda ib, js: (0, 0)),
            pl.BlockSpec((1, co), lambda ib, js: (0, 0)),
        ],
        out_specs=pl.BlockSpec((1, s_t, co), lambda ib, js: (ib, js, 0)),
        out_shape=jax.ShapeDtypeStruct((b, s, co), jnp.float32),
    )(y_bskc, a, c)


_EPS = 1e-5


def _bn_affine(s1, s2, gamma, beta, count):
    mean = s1 / count
    var = s2 / count - mean * mean
    inv = gamma.reshape(1, -1) * jax.lax.rsqrt(var + _EPS)
    return inv, beta.reshape(1, -1) - mean * inv


def _mlp_branch(y1, s1, s2, layers, conv_s_t):
    b, s, k, _ = y1.shape
    count = float(b * s * k)
    a, c = _bn_affine(s1, s2, layers[0][2], layers[0][3], count)
    g = y1
    for (w, bias, gamma, beta) in layers[1:]:
        y, t1, t2 = _conv(g, w.T, bias.reshape(1, -1), a, c,
                          act=True, s_t=conv_s_t)
        a, c = _bn_affine(t1, t2, gamma, beta, count)
        g = y
    return _pool(g, a, c, conv_s_t)


def _sa_layer(xyz_cn, xyz_nc, points_nc, npoint, branches, group_s_t):
    new_c3 = _fps(xyz_cn, npoint)
    new_sn3 = jnp.transpose(new_c3, (0, 2, 1))
    feats = jnp.concatenate([points_nc, xyz_nc], axis=2)
    y1a, y1b, s1a, s2a, s1b, s2b = _group_conv1(new_sn3, xyz_cn, feats,
                                                branches, s_t=group_s_t)
    outs = []
    for (radius, k, layers), y1, s1, s2 in (
            (branches[0], y1a, s1a, s2a), (branches[1], y1b, s1b, s2b)):
        conv_s_t = min(npoint, 1024 // k)
        outs.append(_mlp_branch(y1, s1, s2, layers, conv_s_t))
    return new_c3, new_sn3, jnp.concatenate(outs, axis=2)


def kernel(xyz, params):
    xyz = xyz.astype(jnp.float32)
    xyz_nc = jnp.transpose(xyz, (0, 2, 1))

    sa1 = [(0.05, 16, params['sa1'][0]), (0.1, 32, params['sa1'][1])]
    l1_c3, l1_nc3, l1_points = _sa_layer(xyz, xyz_nc, xyz_nc, 512, sa1,
                                         group_s_t=16)

    sa2 = [(0.1, 16, params['sa2'][0]), (0.2, 32, params['sa2'][1])]
    l2_c3, _, l2_points = _sa_layer(l1_c3, l1_nc3, l1_points, 256, sa2,
                                    group_s_t=32)

    return l2_c3, jnp.transpose(l2_points, (0, 2, 1))
